# baseline (speedup 1.0000x reference)
import functools

import jax
import jax.numpy as jnp
from jax import lax
from jax.experimental import pallas as pl
from jax.experimental.pallas import tpu as pltpu
from jax.experimental.pallas import tpu_sc as plsc

DIM = 1024
HIDDEN = 2816
E = 8
T = 32
HBLK = 1408
LANES = 16


def _sc_routing_kernel(x_hbm, gate_hbm, out_hbm, xv, gv, ov):
    t = lax.axis_index("s") * 2 + lax.axis_index("c")
    pltpu.sync_copy(x_hbm.at[t], xv)
    pltpu.sync_copy(gate_hbm, gv)

    lane = lax.broadcasted_iota(jnp.int32, (LANES,), 0)
    logits = jnp.zeros((LANES,), jnp.float32)
    for e in range(E):
        def body(c, acc):
            off = c * LANES
            return acc + xv[pl.ds(off, LANES)] * gv[e, pl.ds(off, LANES)]
        acc = lax.fori_loop(0, DIM // LANES, body, jnp.zeros((LANES,), jnp.float32))
        logits = jnp.where(lane == e, jnp.sum(acc), logits)

    valid = lane < E
    logits = jnp.where(valid, logits, -1e30)
    p = jnp.exp(logits - jnp.max(logits))
    p = jnp.where(valid, p, 0.0)
    p = p / jnp.sum(p)
    m1 = jnp.max(p)
    i1 = jnp.min(jnp.where(p == m1, lane, LANES))
    first = lane == i1
    pm = jnp.where(first, -1.0, p)
    m2 = jnp.max(pm)
    i2 = jnp.min(jnp.where(pm == m2, lane, LANES))
    second = lane == i2
    sel = jnp.logical_or(first, second)
    ov[...] = jnp.where(sel, p, 0.0) / (m1 + m2)
    pltpu.sync_copy(ov, out_hbm.at[t])


_sc_routing = functools.partial(
    pl.kernel,
    mesh=plsc.VectorSubcoreMesh(core_axis_name="c", subcore_axis_name="s"),
    out_type=jax.ShapeDtypeStruct((T, LANES), jnp.float32),
    scratch_types=[
        pltpu.VMEM((DIM,), jnp.float32),
        pltpu.VMEM((E, DIM), jnp.float32),
        pltpu.VMEM((LANES,), jnp.float32),
    ],
    compiler_params=pltpu.CompilerParams(needs_layout_passes=False),
)


def _ffn_kernel(x_ref, wmat_ref, w1_ref, w3_ref, w2_ref, out_ref):
    e = pl.program_id(0)
    h = pl.program_id(1)
    x = x_ref[...]

    col = jax.lax.broadcasted_iota(jnp.int32, (T, LANES), 1) == e
    wcol = jnp.sum(jnp.where(col, wmat_ref[...], 0.0), axis=-1, keepdims=True)

    h1 = jax.lax.dot_general(
        x, w1_ref[0], (((1,), (1,)), ((), ())), preferred_element_type=jnp.float32
    )
    h3 = jax.lax.dot_general(
        x, w3_ref[0], (((1,), (1,)), ((), ())), preferred_element_type=jnp.float32
    )
    g = (h1 * jax.nn.sigmoid(h1)) * h3
    contrib = jax.lax.dot_general(
        g, w2_ref[0], (((1,), (1,)), ((), ())), preferred_element_type=jnp.float32
    )

    @pl.when(jnp.logical_and(e == 0, h == 0))
    def _init():
        out_ref[...] = jnp.zeros_like(out_ref)

    out_ref[...] += contrib * wcol


@jax.jit
def kernel(x, gate_w, w1, w2, w3):
    wmat = _sc_routing(_sc_routing_kernel)(x, gate_w)
    grid = (E, HIDDEN // HBLK)
    return pl.pallas_call(
        _ffn_kernel,
        grid=grid,
        in_specs=[
            pl.BlockSpec((T, DIM), lambda e, h: (0, 0)),
            pl.BlockSpec((T, LANES), lambda e, h: (0, 0)),
            pl.BlockSpec((1, HBLK, DIM), lambda e, h: (e, h, 0)),
            pl.BlockSpec((1, HBLK, DIM), lambda e, h: (e, h, 0)),
            pl.BlockSpec((1, DIM, HBLK), lambda e, h: (e, 0, h)),
        ],
        out_specs=pl.BlockSpec((T, DIM), lambda e, h: (0, 0)),
        out_shape=jax.ShapeDtypeStruct((T, DIM), jnp.float32),
    )(x, wmat, w1, w3, w2)


# speedup vs baseline: 1.2738x; 1.2738x over previous
import jax
import jax.numpy as jnp
from jax.experimental import pallas as pl
from jax.experimental.pallas import tpu as pltpu

DIM = 1024
HIDDEN = 2816
E = 8
T = 32
HBLK = 1408


def _routing_weights(x, gate_w):
    p = jax.lax.dot_general(
        x, gate_w, (((1,), (1,)), ((), ())), preferred_element_type=jnp.float32
    )
    p = p - jnp.max(p, axis=-1, keepdims=True)
    p = jnp.exp(p)
    p = p / jnp.sum(p, axis=-1, keepdims=True)
    idx = jax.lax.broadcasted_iota(jnp.int32, p.shape, 1)
    m1 = jnp.max(p, axis=-1, keepdims=True)
    i1 = jnp.min(jnp.where(p == m1, idx, E), axis=-1, keepdims=True)
    first = idx == i1
    p_masked = jnp.where(first, -1.0, p)
    m2 = jnp.max(p_masked, axis=-1, keepdims=True)
    i2 = jnp.min(jnp.where(p_masked == m2, idx, E), axis=-1, keepdims=True)
    second = idx == i2
    sel = jnp.logical_or(first, second)
    return jnp.where(sel, p, 0.0) / (m1 + m2)


def _ffn_kernel(x_ref, gate_ref, w1_ref, w3_ref, w2_ref, out_ref, wmat_ref):
    e = pl.program_id(0)
    h = pl.program_id(1)
    x = x_ref[...]

    @pl.when(jnp.logical_and(e == 0, h == 0))
    def _init():
        wmat_ref[...] = _routing_weights(x, gate_ref[...])
        out_ref[...] = jnp.zeros_like(out_ref)

    col = jax.lax.broadcasted_iota(jnp.int32, (T, E), 1) == e
    wcol = jnp.sum(jnp.where(col, wmat_ref[...], 0.0), axis=-1, keepdims=True)

    h1 = jax.lax.dot_general(
        x, w1_ref[0], (((1,), (1,)), ((), ())), preferred_element_type=jnp.float32
    )
    h3 = jax.lax.dot_general(
        x, w3_ref[0], (((1,), (1,)), ((), ())), preferred_element_type=jnp.float32
    )
    g = (h1 * jax.nn.sigmoid(h1)) * h3
    contrib = jax.lax.dot_general(
        g, w2_ref[0], (((1,), (1,)), ((), ())), preferred_element_type=jnp.float32
    )
    out_ref[...] += contrib * wcol


@jax.jit
def kernel(x, gate_w, w1, w2, w3):
    grid = (E, HIDDEN // HBLK)
    return pl.pallas_call(
        _ffn_kernel,
        grid=grid,
        in_specs=[
            pl.BlockSpec((T, DIM), lambda e, h: (0, 0)),
            pl.BlockSpec((E, DIM), lambda e, h: (0, 0)),
            pl.BlockSpec((1, HBLK, DIM), lambda e, h: (e, h, 0)),
            pl.BlockSpec((1, HBLK, DIM), lambda e, h: (e, h, 0)),
            pl.BlockSpec((1, DIM, HBLK), lambda e, h: (e, 0, h)),
        ],
        out_specs=pl.BlockSpec((T, DIM), lambda e, h: (0, 0)),
        out_shape=jax.ShapeDtypeStruct((T, DIM), jnp.float32),
        scratch_shapes=[pltpu.VMEM((T, E), jnp.float32)],
    )(x, gate_w, w1, w3, w2)


# speedup vs baseline: 1.2827x; 1.0070x over previous
import jax
import jax.numpy as jnp
from jax.experimental import pallas as pl
from jax.experimental.pallas import tpu as pltpu

DIM = 1024
HIDDEN = 2816
E = 8
T = 32
HBLK = 1408


def _routing_weights(x, gate_w):
    p = jax.lax.dot_general(
        x, gate_w, (((1,), (1,)), ((), ())), preferred_element_type=jnp.float32
    )
    p = p - jnp.max(p, axis=-1, keepdims=True)
    p = jnp.exp(p)
    p = p / jnp.sum(p, axis=-1, keepdims=True)
    idx = jax.lax.broadcasted_iota(jnp.int32, p.shape, 1)
    m1 = jnp.max(p, axis=-1, keepdims=True)
    i1 = jnp.min(jnp.where(p == m1, idx, E), axis=-1, keepdims=True)
    first = idx == i1
    p_masked = jnp.where(first, -1.0, p)
    m2 = jnp.max(p_masked, axis=-1, keepdims=True)
    i2 = jnp.min(jnp.where(p_masked == m2, idx, E), axis=-1, keepdims=True)
    second = idx == i2
    sel = jnp.logical_or(first, second)
    return jnp.where(sel, p, 0.0) / (m1 + m2)


def _ffn_kernel(x_ref, gate_ref, w1_ref, w3_ref, w2_ref, out_ref):
    e = pl.program_id(0)
    h = pl.program_id(1)
    x = x_ref[...]

    @pl.when(jnp.logical_and(e == 0, h == 0))
    def _init():
        out_ref[...] = jnp.zeros_like(out_ref)

    wmat = _routing_weights(x, gate_ref[...])
    col = jax.lax.broadcasted_iota(jnp.int32, (T, E), 1) == e
    wcol = jnp.sum(jnp.where(col, wmat, 0.0), axis=-1, keepdims=True)

    h1 = jax.lax.dot_general(
        x, w1_ref[0], (((1,), (1,)), ((), ())), preferred_element_type=jnp.float32
    )
    h3 = jax.lax.dot_general(
        x, w3_ref[0], (((1,), (1,)), ((), ())), preferred_element_type=jnp.float32
    )
    g = (h1 * jax.nn.sigmoid(h1)) * h3
    contrib = jax.lax.dot_general(
        g, w2_ref[0], (((1,), (1,)), ((), ())), preferred_element_type=jnp.float32
    )
    out_ref[...] += contrib * wcol


@jax.jit
def kernel(x, gate_w, w1, w2, w3):
    grid = (E, HIDDEN // HBLK)
    return pl.pallas_call(
        _ffn_kernel,
        grid=grid,
        in_specs=[
            pl.BlockSpec((T, DIM), lambda e, h: (0, 0)),
            pl.BlockSpec((E, DIM), lambda e, h: (0, 0)),
            pl.BlockSpec((1, HBLK, DIM), lambda e, h: (e, h, 0)),
            pl.BlockSpec((1, HBLK, DIM), lambda e, h: (e, h, 0)),
            pl.BlockSpec((1, DIM, HBLK), lambda e, h: (e, 0, h)),
        ],
        out_specs=pl.BlockSpec((T, DIM), lambda e, h: (0, 0)),
        out_shape=jax.ShapeDtypeStruct((T, DIM), jnp.float32),
    )(x, gate_w, w1, w3, w2)
